# Initial kernel scaffold; baseline (speedup 1.0000x reference)
import functools

import jax
import jax.numpy as jnp
from jax import lax
from jax.experimental import pallas as pl
from jax.experimental.pallas import tpu as pltpu
from jax.experimental.pallas import tpu_sc as plsc


def _sc_gather3(state_emb, act_emb_self, act_emb_other, sidx, aidx, oidx):
    B = sidx.shape[0]
    D = state_emb.shape[1]
    info = plsc.get_sparse_core_info()
    NC, NS = info.num_cores, info.num_subcores
    NW = NC * NS
    assert B % (8 * NW) == 0
    b_per_w = B // NW

    mesh = plsc.VectorSubcoreMesh(core_axis_name="c", subcore_axis_name="s")
    row_t = jax.ShapeDtypeStruct((B, D), jnp.float32)

    @functools.partial(
        pl.kernel,
        mesh=mesh,
        out_type=[row_t, row_t, row_t],
        scratch_types=[
            pltpu.VMEM((b_per_w,), jnp.int32),
            pltpu.VMEM((b_per_w, D), jnp.float32),
            pltpu.SemaphoreType.DMA,
        ],
    )
    def gather_k(state_hbm, aself_hbm, aother_hbm, sidx_hbm, aidx_hbm,
                 oidx_hbm, out_s, out_a, out_o, idx_v, rows_v, sem):
        wid = lax.axis_index("s") * NC + lax.axis_index("c")
        base = wid * b_per_w

        def one(table_hbm, idx_hbm, out_hbm):
            pltpu.sync_copy(idx_hbm.at[pl.ds(base, b_per_w)], idx_v)
            pltpu.async_copy(table_hbm.at[idx_v], rows_v, sem).wait()
            pltpu.sync_copy(rows_v, out_hbm.at[pl.ds(base, b_per_w)])

        one(state_hbm, sidx_hbm, out_s)
        one(aself_hbm, aidx_hbm, out_a)
        one(aother_hbm, oidx_hbm, out_o)

    return gather_k(state_emb, act_emb_self, act_emb_other, sidx, aidx, oidx)


def _tc_mlp(S, A, O, W1, b1, w2row, b2, block_m):
    B, D = S.shape
    n_blocks = B // block_m

    def body(s_ref, a_ref, o_ref, w1_ref, b1_ref, w2_ref, b2_ref, out_ref):
        h = jnp.dot(s_ref[...], w1_ref[0:D, :], preferred_element_type=jnp.float32)
        h = h + jnp.dot(a_ref[...], w1_ref[D:2 * D, :], preferred_element_type=jnp.float32)
        h = h + jnp.dot(o_ref[...], w1_ref[2 * D:3 * D, :], preferred_element_type=jnp.float32)
        h = jnp.maximum(h + b1_ref[...], 0.0)
        r = jnp.sum(h * w2_ref[...], axis=1) + b2_ref[0, 0]
        out_ref[...] = r[None, :]

    out = pl.pallas_call(
        body,
        grid=(n_blocks,),
        in_specs=[
            pl.BlockSpec((block_m, D), lambda i: (i, 0)),
            pl.BlockSpec((block_m, D), lambda i: (i, 0)),
            pl.BlockSpec((block_m, D), lambda i: (i, 0)),
            pl.BlockSpec((3 * D, D), lambda i: (0, 0)),
            pl.BlockSpec((1, D), lambda i: (0, 0)),
            pl.BlockSpec((1, D), lambda i: (0, 0)),
            pl.BlockSpec(memory_space=pltpu.SMEM),
        ],
        out_specs=pl.BlockSpec((1, block_m), lambda i: (i, 0)),
        out_shape=jax.ShapeDtypeStruct((n_blocks, block_m), jnp.float32),
        compiler_params=pltpu.CompilerParams(
            dimension_semantics=("arbitrary",),
        ),
    )(S, A, O, W1, b1, w2row, b2)
    return out.reshape(B)


def kernel(state_indices, joint_actions, state_emb, act_emb_self,
           act_emb_other, W1, b1, W2, b2):
    sidx = state_indices.astype(jnp.int32)
    aidx = joint_actions[:, 0].astype(jnp.int32)
    oidx = joint_actions[:, 1].astype(jnp.int32)

    S, A, O = _sc_gather3(state_emb, act_emb_self, act_emb_other,
                          sidx, aidx, oidx)

    b1r = b1.reshape(1, -1)
    w2row = W2.reshape(1, -1)
    b2r = b2.reshape(1, 1)
    return _tc_mlp(S, A, O, W1, b1r, w2row, b2r, block_m=1024)


# speedup vs baseline: 2.7805x; 2.7805x over previous
import functools

import jax
import jax.numpy as jnp
from jax import lax
from jax.experimental import pallas as pl
from jax.experimental.pallas import tpu as pltpu
from jax.experimental.pallas import tpu_sc as plsc


def _sc_gather3(state_emb, act_emb_self, act_emb_other, sidx, aidx, oidx):
    B = sidx.shape[0]
    D = state_emb.shape[1]
    info = plsc.get_sparse_core_info()
    NC, NS = info.num_cores, info.num_subcores
    NW = NC * NS
    assert B % (8 * NW) == 0
    b_per_w = B // NW

    mesh = plsc.VectorSubcoreMesh(core_axis_name="c", subcore_axis_name="s")
    row_t = jax.ShapeDtypeStruct((B, D), jnp.float32)

    @functools.partial(
        pl.kernel,
        mesh=mesh,
        out_type=[row_t, row_t, row_t],
        scratch_types=[
            pltpu.VMEM((b_per_w,), jnp.int32),
            pltpu.VMEM((b_per_w, D), jnp.float32),
            pltpu.SemaphoreType.DMA,
        ],
    )
    def gather_k(state_hbm, aself_hbm, aother_hbm, sidx_hbm, aidx_hbm,
                 oidx_hbm, out_s, out_a, out_o, idx_v, rows_v, sem):
        wid = lax.axis_index("s") * NC + lax.axis_index("c")
        base = wid * b_per_w

        def one(table_hbm, idx_hbm, out_hbm):
            pltpu.sync_copy(idx_hbm.at[pl.ds(base, b_per_w)], idx_v)
            pltpu.async_copy(table_hbm.at[idx_v], rows_v, sem).wait()
            pltpu.sync_copy(rows_v, out_hbm.at[pl.ds(base, b_per_w)])

        one(state_hbm, sidx_hbm, out_s)
        one(aself_hbm, aidx_hbm, out_a)
        one(aother_hbm, oidx_hbm, out_o)

    return gather_k(state_emb, act_emb_self, act_emb_other, sidx, aidx, oidx)


def _tc_mlp(S, A, O, W1, b1, w2row, b2, block_m):
    B, D = S.shape
    n_blocks = B // block_m

    def body(s_ref, a_ref, o_ref, w1_ref, b1_ref, w2_ref, b2_ref, out_ref):
        h = jnp.dot(s_ref[...], w1_ref[0:D, :], preferred_element_type=jnp.float32)
        h = h + jnp.dot(a_ref[...], w1_ref[D:2 * D, :], preferred_element_type=jnp.float32)
        h = h + jnp.dot(o_ref[...], w1_ref[2 * D:3 * D, :], preferred_element_type=jnp.float32)
        h = jnp.maximum(h + b1_ref[...], 0.0)
        r = jnp.sum(h * w2_ref[...], axis=1) + b2_ref[0, 0]
        out_ref[...] = r[None, None, :]

    out = pl.pallas_call(
        body,
        grid=(n_blocks,),
        in_specs=[
            pl.BlockSpec((block_m, D), lambda i: (i, 0)),
            pl.BlockSpec((block_m, D), lambda i: (i, 0)),
            pl.BlockSpec((block_m, D), lambda i: (i, 0)),
            pl.BlockSpec((3 * D, D), lambda i: (0, 0)),
            pl.BlockSpec((1, D), lambda i: (0, 0)),
            pl.BlockSpec((1, D), lambda i: (0, 0)),
            pl.BlockSpec(memory_space=pltpu.SMEM),
        ],
        out_specs=pl.BlockSpec((1, 1, block_m), lambda i: (i, 0, 0)),
        out_shape=jax.ShapeDtypeStruct((n_blocks, 1, block_m), jnp.float32),
        compiler_params=pltpu.CompilerParams(
            dimension_semantics=("arbitrary",),
        ),
    )(S, A, O, W1, b1, w2row, b2)
    return out.reshape(B)


def kernel(state_indices, joint_actions, state_emb, act_emb_self,
           act_emb_other, W1, b1, W2, b2):
    sidx = state_indices.astype(jnp.int32)
    aidx = joint_actions[:, 0].astype(jnp.int32)
    oidx = joint_actions[:, 1].astype(jnp.int32)

    S, A, O = _sc_gather3(state_emb, act_emb_self, act_emb_other,
                          sidx, aidx, oidx)

    b1r = b1.reshape(1, -1)
    w2row = W2.reshape(1, -1)
    b2r = b2.reshape(1, 1)
    return _tc_mlp(S, A, O, W1, b1r, w2row, b2r, block_m=1024)


# speedup vs baseline: 2.9218x; 1.0508x over previous
import functools

import jax
import jax.numpy as jnp
from jax import lax
from jax.experimental import pallas as pl
from jax.experimental.pallas import tpu as pltpu
from jax.experimental.pallas import tpu_sc as plsc


def _sc_gather3(state_emb, act_emb_self, act_emb_other, sidx, aidx, oidx):
    B = sidx.shape[0]
    D = state_emb.shape[1]
    info = plsc.get_sparse_core_info()
    NC, NS = info.num_cores, info.num_subcores
    NW = NC * NS
    assert B % (16 * NW) == 0
    b_per_w = B // NW
    ch = b_per_w // 2

    mesh = plsc.VectorSubcoreMesh(core_axis_name="c", subcore_axis_name="s")
    row_t = jax.ShapeDtypeStruct((B, D), jnp.float32)

    @functools.partial(
        pl.kernel,
        mesh=mesh,
        out_type=[row_t, row_t, row_t],
        scratch_types=[
            pltpu.VMEM((3 * b_per_w,), jnp.int32),
            pltpu.VMEM((ch, D), jnp.float32),
            pltpu.VMEM((ch, D), jnp.float32),
            pltpu.VMEM((ch, D), jnp.float32),
            pltpu.SemaphoreType.DMA,
            pltpu.SemaphoreType.DMA,
            pltpu.SemaphoreType.DMA,
            pltpu.SemaphoreType.DMA,
            pltpu.SemaphoreType.DMA,
            pltpu.SemaphoreType.DMA,
            pltpu.SemaphoreType.DMA,
        ],
    )
    def gather_k(state_hbm, aself_hbm, aother_hbm, sidx_hbm, aidx_hbm,
                 oidx_hbm, out_s, out_a, out_o, idx_v, buf0, buf1, buf2,
                 g0, g1, g2, w0, w1, w2, isem):
        wid = lax.axis_index("s") * NC + lax.axis_index("c")
        base = wid * b_per_w

        tables = [state_hbm, aself_hbm, aother_hbm]
        idxs = [sidx_hbm, aidx_hbm, oidx_hbm]
        outs = [out_s, out_a, out_o]
        bufs = [buf0, buf1, buf2]
        gsems = [g0, g1, g2]
        wsems = [w0, w1, w2]

        icopies = [
            pltpu.make_async_copy(
                idxs[t].at[pl.ds(base, b_per_w)],
                idx_v.at[pl.ds(t * b_per_w, b_per_w)], isem)
            for t in range(3)
        ]
        for c in icopies:
            c.start()

        gathers = [None] * 6
        writes = [None] * 6

        for c in icopies:
            c.wait()

        for c in range(3):
            t, h = c // 2, c % 2
            gathers[c] = pltpu.make_async_copy(
                tables[t].at[idx_v.at[pl.ds(t * b_per_w + h * ch, ch)]],
                bufs[c % 3], gsems[c % 3])
            gathers[c].start()

        for c in range(6):
            t, h = c // 2, c % 2
            gathers[c].wait()
            writes[c] = pltpu.make_async_copy(
                bufs[c % 3],
                outs[t].at[pl.ds(base + h * ch, ch)], wsems[c % 3])
            writes[c].start()
            nc = c + 3
            if nc < 6:
                tn, hn = nc // 2, nc % 2
                writes[c].wait()
                gathers[nc] = pltpu.make_async_copy(
                    tables[tn].at[idx_v.at[pl.ds(tn * b_per_w + hn * ch, ch)]],
                    bufs[nc % 3], gsems[nc % 3])
                gathers[nc].start()

        for c in range(3, 6):
            writes[c].wait()

    return gather_k(state_emb, act_emb_self, act_emb_other, sidx, aidx, oidx)


def _tc_mlp(S, A, O, W1, b1, W2, b2, block_m):
    B, D = S.shape
    n_blocks = B // block_m

    def body(s_ref, a_ref, o_ref, w1_ref, b1_ref, w2_ref, b2_ref, out_ref):
        h = jnp.dot(s_ref[...], w1_ref[0:D, :], preferred_element_type=jnp.float32)
        h = h + jnp.dot(a_ref[...], w1_ref[D:2 * D, :], preferred_element_type=jnp.float32)
        h = h + jnp.dot(o_ref[...], w1_ref[2 * D:3 * D, :], preferred_element_type=jnp.float32)
        h = jnp.maximum(h + b1_ref[...], 0.0)
        out_ref[...] = jnp.dot(h, w2_ref[...], preferred_element_type=jnp.float32) + b2_ref[0]

    out = pl.pallas_call(
        body,
        grid=(n_blocks,),
        in_specs=[
            pl.BlockSpec((block_m, D), lambda i: (i, 0)),
            pl.BlockSpec((block_m, D), lambda i: (i, 0)),
            pl.BlockSpec((block_m, D), lambda i: (i, 0)),
            pl.BlockSpec((3 * D, D), lambda i: (0, 0)),
            pl.BlockSpec((1, D), lambda i: (0, 0)),
            pl.BlockSpec((D, 1), lambda i: (0, 0)),
            pl.BlockSpec(memory_space=pltpu.SMEM),
        ],
        out_specs=pl.BlockSpec((block_m, 1), lambda i: (i, 0)),
        out_shape=jax.ShapeDtypeStruct((B, 1), jnp.float32),
        compiler_params=pltpu.CompilerParams(
            dimension_semantics=("arbitrary",),
        ),
    )(S, A, O, W1, b1, W2, b2)
    return out.reshape(B)


def kernel(state_indices, joint_actions, state_emb, act_emb_self,
           act_emb_other, W1, b1, W2, b2):
    sidx = state_indices.astype(jnp.int32)
    aidx = joint_actions[:, 0].astype(jnp.int32)
    oidx = joint_actions[:, 1].astype(jnp.int32)

    S, A, O = _sc_gather3(state_emb, act_emb_self, act_emb_other,
                          sidx, aidx, oidx)

    b1r = b1.reshape(1, -1)
    return _tc_mlp(S, A, O, W1, b1r, W2, b2, block_m=2048)


# speedup vs baseline: 3.2930x; 1.1271x over previous
import functools

import jax
import jax.numpy as jnp
from jax import lax
from jax.experimental import pallas as pl
from jax.experimental.pallas import tpu as pltpu
from jax.experimental.pallas import tpu_sc as plsc


def _sc_gather3(state_emb, act_emb_self, act_emb_other, sidx, aidx, oidx):
    B = sidx.shape[0]
    D = state_emb.shape[1]
    info = plsc.get_sparse_core_info()
    NC, NS = info.num_cores, info.num_subcores
    NW = NC * NS
    assert B % (16 * NW) == 0
    b_per_w = B // NW
    ch = b_per_w // 2

    mesh = plsc.VectorSubcoreMesh(core_axis_name="c", subcore_axis_name="s")
    row_t = jax.ShapeDtypeStruct((B, D), jnp.float32)

    @functools.partial(
        pl.kernel,
        mesh=mesh,
        out_type=[row_t, row_t, row_t],
        scratch_types=[
            pltpu.VMEM((3 * b_per_w,), jnp.int32),
            pltpu.VMEM((ch, D), jnp.float32),
            pltpu.VMEM((ch, D), jnp.float32),
            pltpu.VMEM((ch, D), jnp.float32),
            pltpu.SemaphoreType.DMA,
            pltpu.SemaphoreType.DMA,
            pltpu.SemaphoreType.DMA,
            pltpu.SemaphoreType.DMA,
            pltpu.SemaphoreType.DMA,
            pltpu.SemaphoreType.DMA,
            pltpu.SemaphoreType.DMA,
        ],
    )
    def gather_k(state_hbm, aself_hbm, aother_hbm, sidx_hbm, aidx_hbm,
                 oidx_hbm, out_s, out_a, out_o, idx_v, buf0, buf1, buf2,
                 g0, g1, g2, w0, w1, w2, isem):
        wid = lax.axis_index("s") * NC + lax.axis_index("c")
        base = wid * b_per_w

        tables = [state_hbm, aself_hbm, aother_hbm]
        idxs = [sidx_hbm, aidx_hbm, oidx_hbm]
        outs = [out_s, out_a, out_o]
        bufs = [buf0, buf1, buf2]
        gsems = [g0, g1, g2]
        wsems = [w0, w1, w2]

        icopies = [
            pltpu.make_async_copy(
                idxs[t].at[pl.ds(base, b_per_w)],
                idx_v.at[pl.ds(t * b_per_w, b_per_w)], isem)
            for t in range(3)
        ]
        for c in icopies:
            c.start()

        gathers = [None] * 6
        writes = [None] * 6

        for c in icopies:
            c.wait()

        for c in range(3):
            t, h = c // 2, c % 2
            gathers[c] = pltpu.make_async_copy(
                tables[t].at[idx_v.at[pl.ds(t * b_per_w + h * ch, ch)]],
                bufs[c % 3], gsems[c % 3])
            gathers[c].start()

        for c in range(6):
            t, h = c // 2, c % 2
            gathers[c].wait()
            writes[c] = pltpu.make_async_copy(
                bufs[c % 3],
                outs[t].at[pl.ds(base + h * ch, ch)], wsems[c % 3])
            writes[c].start()
            nc = c + 3
            if nc < 6:
                tn, hn = nc // 2, nc % 2
                writes[c].wait()
                gathers[nc] = pltpu.make_async_copy(
                    tables[tn].at[idx_v.at[pl.ds(tn * b_per_w + hn * ch, ch)]],
                    bufs[nc % 3], gsems[nc % 3])
                gathers[nc].start()

        for c in range(3, 6):
            writes[c].wait()

    return gather_k(state_emb, act_emb_self, act_emb_other, sidx, aidx, oidx)


def _tc_mlp(S, A, O, W1, b1, W2, b2, block_m):
    B, D = S.shape
    n_blocks = B // block_m

    def body(s_ref, a_ref, o_ref, w1_ref, b1_ref, w2_ref, b2_ref, out_ref):
        h = jnp.dot(s_ref[...], w1_ref[0:D, :], preferred_element_type=jnp.float32)
        h = h + jnp.dot(a_ref[...], w1_ref[D:2 * D, :], preferred_element_type=jnp.float32)
        h = h + jnp.dot(o_ref[...], w1_ref[2 * D:3 * D, :], preferred_element_type=jnp.float32)
        h = jnp.maximum(h + b1_ref[...], 0.0)
        r = lax.dot_general(w2_ref[...], h, (((0,), (1,)), ((), ())),
                            preferred_element_type=jnp.float32)
        out_ref[...] = r + b2_ref[0]

    out = pl.pallas_call(
        body,
        grid=(n_blocks,),
        in_specs=[
            pl.BlockSpec((block_m, D), lambda i: (i, 0)),
            pl.BlockSpec((block_m, D), lambda i: (i, 0)),
            pl.BlockSpec((block_m, D), lambda i: (i, 0)),
            pl.BlockSpec((3 * D, D), lambda i: (0, 0)),
            pl.BlockSpec((1, D), lambda i: (0, 0)),
            pl.BlockSpec((D, 1), lambda i: (0, 0)),
            pl.BlockSpec(memory_space=pltpu.SMEM),
        ],
        out_specs=pl.BlockSpec((1, block_m), lambda i: (0, i)),
        out_shape=jax.ShapeDtypeStruct((1, B), jnp.float32),
        compiler_params=pltpu.CompilerParams(
            dimension_semantics=("arbitrary",),
        ),
    )(S, A, O, W1, b1, W2, b2)
    return out.reshape(B)


def kernel(state_indices, joint_actions, state_emb, act_emb_self,
           act_emb_other, W1, b1, W2, b2):
    sidx = state_indices.astype(jnp.int32)
    aidx = joint_actions[:, 0].astype(jnp.int32)
    oidx = joint_actions[:, 1].astype(jnp.int32)

    S, A, O = _sc_gather3(state_emb, act_emb_self, act_emb_other,
                          sidx, aidx, oidx)

    b1r = b1.reshape(1, -1)
    return _tc_mlp(S, A, O, W1, b1r, W2, b2, block_m=2048)
